# baseline (speedup 1.0000x reference)
import functools

import jax
import jax.numpy as jnp
from jax import lax
from jax.experimental import pallas as pl
from jax.experimental.pallas import tpu as pltpu
import jax.experimental.pallas.tpu_sc as plsc

NC = 2
NS = 16
CH = 128
NBUF = 3


@functools.cache
def _part_call(epw_pad, half, cap):
    nw = NC * NS
    nchcap = cap // CH
    assert cap >= epw_pad + NBUF * CH

    mesh = plsc.VectorSubcoreMesh(core_axis_name="c", subcore_axis_name="s")

    @functools.partial(
        pl.kernel,
        out_type=[
            jax.ShapeDtypeStruct((2, nw, nchcap, CH), jnp.int32),
            jax.ShapeDtypeStruct((2, nw, nchcap, CH), jnp.int32),
            jax.ShapeDtypeStruct((2, nw, 16), jnp.int32),
        ],
        mesh=mesh,
        compiler_params=pltpu.CompilerParams(needs_layout_passes=False),
        scratch_types=[
            pltpu.VMEM((epw_pad,), jnp.int32),
            pltpu.VMEM((epw_pad,), jnp.int32),
            pltpu.VMEM((cap,), jnp.int32),
            pltpu.VMEM((cap,), jnp.int32),
            pltpu.VMEM((cap,), jnp.int32),
            pltpu.VMEM((cap,), jnp.int32),
            pltpu.VMEM((16,), jnp.int32),
            pltpu.VMEM((16,), jnp.int32),
        ],
    )
    def part(srcp_hbm, dstp_hbm, esrc_hbm, edst_hbm, cnts_hbm, src_f, dst_f,
             lo_src, lo_dst, hi_src, hi_dst, cnt_lo, cnt_hi):
        c = lax.axis_index("c")
        s = lax.axis_index("s")
        wid = c * NS + s
        pltpu.sync_copy(srcp_hbm.at[wid], src_f)
        pltpu.sync_copy(dstp_hbm.at[wid], dst_f)

        iota16 = lax.broadcasted_iota(jnp.int32, (16,), 0)
        trash = cap - 16 + iota16
        zero16 = jnp.zeros((16,), jnp.int32)
        padrow = jnp.full((16,), half, jnp.int32)

        def prefill(j, carry):
            lo_src[pl.ds(j * 16, 16)] = zero16
            lo_dst[pl.ds(j * 16, 16)] = padrow
            hi_src[pl.ds(j * 16, 16)] = zero16
            hi_dst[pl.ds(j * 16, 16)] = padrow
            return carry

        lax.fori_loop(0, cap // 16, prefill, 0)

        def step(i, carry):
            clo, chi = carry
            sv = src_f[pl.ds(i * 16, 16)]
            dv = dst_f[pl.ds(i * 16, 16)]
            mlo = dv < half
            mi = mlo.astype(jnp.int32)
            dloc = jnp.where(mlo, dv, dv - half)
            exc = plsc.cumsum(mi) - mi
            plo = jnp.where(mlo, clo + exc, trash)
            phi = jnp.where(mlo, trash, chi + iota16 - exc)
            plsc.store_scatter(lo_src, [plo], sv)
            plsc.store_scatter(lo_dst, [plo], dloc)
            plsc.store_scatter(hi_src, [phi], sv)
            plsc.store_scatter(hi_dst, [phi], dloc)
            nlo = jnp.sum(mi)
            return clo + nlo, chi + (16 - nlo)

        clo, chi = lax.fori_loop(0, epw_pad // 16, step, (0, 0))


        for r in range(nchcap):
            pltpu.sync_copy(lo_src.at[pl.ds(r * CH, CH)],
                            esrc_hbm.at[0, wid, r])
            pltpu.sync_copy(hi_src.at[pl.ds(r * CH, CH)],
                            esrc_hbm.at[1, wid, r])
            pltpu.sync_copy(lo_dst.at[pl.ds(r * CH, CH)],
                            edst_hbm.at[0, wid, r])
            pltpu.sync_copy(hi_dst.at[pl.ds(r * CH, CH)],
                            edst_hbm.at[1, wid, r])

        ncl = lax.div(clo + (CH - 1), CH)
        ncl = lax.div(ncl + (NBUF - 1), NBUF) * NBUF
        nch = lax.div(chi + (CH - 1), CH)
        nch = lax.div(nch + (NBUF - 1), NBUF) * NBUF
        cnt_lo[...] = jnp.full((16,), ncl, jnp.int32)
        cnt_hi[...] = jnp.full((16,), nch, jnp.int32)
        pltpu.sync_copy(cnt_lo, cnts_hbm.at[0, wid])
        pltpu.sync_copy(cnt_hi, cnts_hbm.at[1, wid])

    return part


@functools.cache
def _agg_call(n, d, half, cap):
    nw = NC * NS
    nchcap = cap // CH
    acc_rows = half + CH
    assert acc_rows % (NS * 8) == 0 and half % (NS * 8) == 0
    zrows = acc_rows // NS
    orows = half // NS

    mesh = plsc.VectorSubcoreMesh(core_axis_name="c", subcore_axis_name="s")

    @functools.partial(
        pl.kernel,
        out_type=jax.ShapeDtypeStruct((2, half, d), jnp.float32),
        mesh=mesh,
        compiler_params=pltpu.CompilerParams(needs_layout_passes=False),
        scratch_types=(
            [pltpu.VMEM((nchcap, CH), jnp.int32),
             pltpu.VMEM((nchcap, CH), jnp.int32),
             pltpu.VMEM((16,), jnp.int32),
             pltpu.VMEM((16,), jnp.int32)]
            + [pltpu.VMEM((CH, d), jnp.float32) for _ in range(NBUF)]
            + [pltpu.VMEM_SHARED((acc_rows, d), jnp.float32),
               pltpu.SemaphoreType.DMA]
        ),
    )
    def agg(h_hbm, esrc_hbm, edst_hbm, cnts_hbm, zeros_hbm, out_hbm, src_f,
            dst2d, cnts_v0, cnts_v1, *rest):
        bufs = rest[:NBUF]
        acc = rest[NBUF]
        sem = rest[NBUF + 1]
        c = lax.axis_index("c")
        s = lax.axis_index("s")
        pltpu.sync_copy(zeros_hbm.at[pl.ds(s * zrows, zrows)],
                        acc.at[pl.ds(s * zrows, zrows)])
        pltpu.sync_copy(cnts_hbm.at[c, 2 * s], cnts_v0)
        pltpu.sync_copy(cnts_hbm.at[c, 2 * s + 1], cnts_v1)
        plsc.subcore_barrier()
        nch_w = [jnp.max(cnts_v0[...]), jnp.max(cnts_v1[...])]

        for wsub in range(2):
            w = 2 * s + wsub
            pltpu.sync_copy(esrc_hbm.at[c, w], src_f)
            pltpu.sync_copy(edst_hbm.at[c, w], dst2d)

            nch = nch_w[wsub]

            def group(g3, carry):
                @pl.when(g3 * NBUF < nch)
                def _():
                    descs = []
                    for b in range(NBUF):
                        ch = g3 * NBUF + b
                        descs.append(pltpu.async_copy(
                            h_hbm.at[src_f.at[ch]], bufs[b], sem))
                    for b in range(NBUF):
                        descs[b].wait()
                    for b in range(NBUF):
                        pltpu.sync_copy(
                            bufs[b], acc.at[dst2d.at[g3 * NBUF + b]],
                            add=True)
                return carry

            lax.fori_loop(0, nchcap // NBUF, group, 0)

        plsc.subcore_barrier()
        pltpu.sync_copy(acc.at[pl.ds(s * orows, orows)],
                        out_hbm.at[c, pl.ds(s * orows, orows)])

    return agg


def _mlp_body(scale_ref, h_ref, agg_ref, w1_ref, b1_ref, g_ref, be_ref, w2_ref,
              b2_ref, out_ref):
    n = h_ref.shape[0]
    aggcat = jnp.concatenate([agg_ref[0], agg_ref[1]], axis=0)
    z = h_ref[:] * scale_ref[0, 0] + aggcat[:n]
    z1 = jnp.dot(z, w1_ref[:], preferred_element_type=jnp.float32) + b1_ref[:]
    z1 = jnp.maximum(z1, 0.0)
    mu = jnp.mean(z1, axis=0, keepdims=True)
    cen = z1 - mu
    var = jnp.mean(cen * cen, axis=0, keepdims=True)
    z2 = cen * lax.rsqrt(var + 1e-5) * g_ref[:] + be_ref[:]
    z2 = jnp.maximum(z2, 0.0)
    z3 = jnp.dot(z2, w2_ref[:], preferred_element_type=jnp.float32) + b2_ref[:]
    out_ref[:] = jnp.maximum(z3, 0.0)


@functools.cache
def _mlp_call(n, d, h):
    return pl.pallas_call(
        _mlp_body,
        out_shape=jax.ShapeDtypeStruct((n, h), jnp.float32),
        in_specs=[pl.BlockSpec(memory_space=pltpu.SMEM)]
        + [pl.BlockSpec(memory_space=pltpu.VMEM)] * 8,
        out_specs=pl.BlockSpec(memory_space=pltpu.VMEM),
    )


def _final_body(batch_ref, h_ref, w1_ref, b1_ref, w2_ref, b2_ref, out_ref, *,
                g):
    n = h_ref.shape[0]
    gids = lax.broadcasted_iota(jnp.int32, (g, n), 0)
    onehot = (batch_ref[:] == gids).astype(jnp.float32)
    gp = jnp.dot(onehot, h_ref[:], preferred_element_type=jnp.float32)
    g1 = jnp.dot(gp, w1_ref[:], preferred_element_type=jnp.float32) + b1_ref[:]
    g1 = jnp.maximum(g1, 0.0)
    out_ref[:] = (jnp.dot(g1, w2_ref[:], preferred_element_type=jnp.float32)
                  + b2_ref[:])


@functools.cache
def _final_call(n, h, o, g):
    return pl.pallas_call(
        functools.partial(_final_body, g=g),
        out_shape=jax.ShapeDtypeStruct((g, o), jnp.float32),
        in_specs=[pl.BlockSpec(memory_space=pltpu.VMEM)] * 6,
        out_specs=pl.BlockSpec(memory_space=pltpu.VMEM),
    )


def kernel(x, edge_index, batch, eps, W1, b1, gamma, beta, W2, b2, lin1_W,
           lin1_b, lin2_W, lin2_b):
    n, d = x.shape
    e = edge_index.shape[1]
    nlayers, _, hdim = W1.shape
    odim = lin2_W.shape[1]
    g = 64

    nw = NC * NS
    epw = e // nw
    epw_pad = ((epw + 15) // 16) * 16
    half = ((n + 2 * NS * 8 - 1) // (2 * NS * 8)) * (NS * 8)
    src = edge_index[0].reshape(nw, epw)
    dst = edge_index[1].reshape(nw, epw)
    pad = epw_pad - epw
    if pad:
        src = jnp.pad(src, ((0, 0), (0, pad)))
        dst = jnp.pad(dst, ((0, 0), (0, pad)), constant_values=half)
    cap = ((epw_pad + NBUF * CH) + NBUF * CH - 1) // (NBUF * CH) * (NBUF * CH)

    esrc, edst, cnts = _part_call(epw_pad, half, cap)(src, dst)
    zeros = jnp.zeros((half + CH, d), jnp.float32)
    agg_fn = _agg_call(n, d, half, cap)
    mlp_fn = _mlp_call(n, d, hdim)

    h = x
    for i in range(nlayers):
        agg = agg_fn(h, esrc, edst, cnts, zeros)
        scale = (1.0 + eps[i]).reshape(1, 1)
        h = mlp_fn(scale, h, agg, W1[i], b1[i].reshape(1, hdim),
                   gamma[i].reshape(1, hdim), beta[i].reshape(1, hdim), W2[i],
                   b2[i].reshape(1, hdim))

    return _final_call(n, hdim, odim, g)(
        batch.reshape(1, n), h, lin1_W, lin1_b.reshape(1, odim), lin2_W,
        lin2_b.reshape(1, odim))


# speedup vs baseline: 3.2220x; 3.2220x over previous
import functools

import jax
import jax.numpy as jnp
from jax import lax
from jax.experimental import pallas as pl
from jax.experimental.pallas import tpu as pltpu
import jax.experimental.pallas.tpu_sc as plsc

NC = 2
NS = 16
CH = 128
NBUF = 3


@functools.cache
def _part_call(epw_pad, half, cap):
    nw = NC * NS
    nchcap = cap // CH
    assert cap >= epw_pad + NBUF * CH

    mesh = plsc.VectorSubcoreMesh(core_axis_name="c", subcore_axis_name="s")

    @functools.partial(
        pl.kernel,
        out_type=[
            jax.ShapeDtypeStruct((2, nw, nchcap, CH), jnp.int32),
            jax.ShapeDtypeStruct((2, nw, nchcap, CH), jnp.int32),
            jax.ShapeDtypeStruct((2, nw, 16), jnp.int32),
        ],
        mesh=mesh,
        compiler_params=pltpu.CompilerParams(needs_layout_passes=False),
        scratch_types=[
            pltpu.VMEM((epw_pad,), jnp.int32),
            pltpu.VMEM((epw_pad,), jnp.int32),
            pltpu.VMEM((cap,), jnp.int32),
            pltpu.VMEM((cap,), jnp.int32),
            pltpu.VMEM((cap,), jnp.int32),
            pltpu.VMEM((cap,), jnp.int32),
            pltpu.VMEM((16,), jnp.int32),
            pltpu.VMEM((16,), jnp.int32),
        ],
    )
    def part(srcp_hbm, dstp_hbm, esrc_hbm, edst_hbm, cnts_hbm, src_f, dst_f,
             lo_src, lo_dst, hi_src, hi_dst, cnt_lo, cnt_hi):
        c = lax.axis_index("c")
        s = lax.axis_index("s")
        wid = c * NS + s
        pltpu.sync_copy(srcp_hbm.at[wid], src_f)
        pltpu.sync_copy(dstp_hbm.at[wid], dst_f)

        iota16 = lax.broadcasted_iota(jnp.int32, (16,), 0)
        trash = cap - 16 + iota16
        zero16 = jnp.zeros((16,), jnp.int32)
        padrow = jnp.full((16,), half, jnp.int32)

        def prefill(j, carry):
            spread = lax.rem(j * 16 + iota16, 1024)
            lo_src[pl.ds(j * 16, 16)] = spread
            lo_dst[pl.ds(j * 16, 16)] = padrow
            hi_src[pl.ds(j * 16, 16)] = spread
            hi_dst[pl.ds(j * 16, 16)] = padrow
            return carry

        lax.fori_loop(0, cap // 16, prefill, 0)

        def step(i, carry):
            clo, chi = carry
            sv = src_f[pl.ds(i * 16, 16)]
            dv = dst_f[pl.ds(i * 16, 16)]
            mlo = dv < half
            mi = mlo.astype(jnp.int32)
            dloc = jnp.where(mlo, dv, dv - half)
            exc = plsc.cumsum(mi) - mi
            plo = jnp.where(mlo, clo + exc, trash)
            phi = jnp.where(mlo, trash, chi + iota16 - exc)
            plsc.store_scatter(lo_src, [plo], sv)
            plsc.store_scatter(lo_dst, [plo], dloc)
            plsc.store_scatter(hi_src, [phi], sv)
            plsc.store_scatter(hi_dst, [phi], dloc)
            nlo = jnp.sum(mi)
            return clo + nlo, chi + (16 - nlo)

        clo, chi = lax.fori_loop(0, epw_pad // 16, step, (0, 0))


        for r in range(nchcap):
            pltpu.sync_copy(lo_src.at[pl.ds(r * CH, CH)],
                            esrc_hbm.at[0, wid, r])
            pltpu.sync_copy(hi_src.at[pl.ds(r * CH, CH)],
                            esrc_hbm.at[1, wid, r])
            pltpu.sync_copy(lo_dst.at[pl.ds(r * CH, CH)],
                            edst_hbm.at[0, wid, r])
            pltpu.sync_copy(hi_dst.at[pl.ds(r * CH, CH)],
                            edst_hbm.at[1, wid, r])

        ncl = lax.div(clo + (CH - 1), CH)
        ncl = lax.div(ncl + (NBUF - 1), NBUF) * NBUF
        nch = lax.div(chi + (CH - 1), CH)
        nch = lax.div(nch + (NBUF - 1), NBUF) * NBUF
        cnt_lo[...] = jnp.full((16,), ncl, jnp.int32)
        cnt_hi[...] = jnp.full((16,), nch, jnp.int32)
        pltpu.sync_copy(cnt_lo, cnts_hbm.at[0, wid])
        pltpu.sync_copy(cnt_hi, cnts_hbm.at[1, wid])

    return part


@functools.cache
def _agg_call(n, d, half, cap):
    nw = NC * NS
    nchcap = cap // CH
    acc_rows = half + CH
    assert acc_rows % (NS * 8) == 0 and half % (NS * 8) == 0
    zrows = acc_rows // NS
    orows = half // NS

    mesh = plsc.VectorSubcoreMesh(core_axis_name="c", subcore_axis_name="s")

    @functools.partial(
        pl.kernel,
        out_type=jax.ShapeDtypeStruct((2, half, d), jnp.float32),
        mesh=mesh,
        compiler_params=pltpu.CompilerParams(needs_layout_passes=False),
        scratch_types=(
            [pltpu.VMEM((nchcap, CH), jnp.int32),
             pltpu.VMEM((nchcap, CH), jnp.int32),
             pltpu.VMEM((16,), jnp.int32),
             pltpu.VMEM((16,), jnp.int32)]
            + [pltpu.VMEM((CH, d), jnp.float32) for _ in range(NBUF)]
            + [pltpu.VMEM_SHARED((acc_rows, d), jnp.float32),
               pltpu.SemaphoreType.DMA]
        ),
    )
    def agg(h_hbm, esrc_hbm, edst_hbm, cnts_hbm, zeros_hbm, out_hbm, src_f,
            dst2d, cnts_v0, cnts_v1, *rest):
        bufs = rest[:NBUF]
        acc = rest[NBUF]
        sem = rest[NBUF + 1]
        c = lax.axis_index("c")
        s = lax.axis_index("s")
        pltpu.sync_copy(zeros_hbm.at[pl.ds(s * zrows, zrows)],
                        acc.at[pl.ds(s * zrows, zrows)])
        pltpu.sync_copy(cnts_hbm.at[c, 2 * s], cnts_v0)
        pltpu.sync_copy(cnts_hbm.at[c, 2 * s + 1], cnts_v1)
        plsc.subcore_barrier()
        nch_w = [jnp.max(cnts_v0[...]), jnp.max(cnts_v1[...])]

        for wsub in range(2):
            w = 2 * s + wsub
            pltpu.sync_copy(esrc_hbm.at[c, w], src_f)
            pltpu.sync_copy(edst_hbm.at[c, w], dst2d)

            nch = nch_w[wsub]

            def group(g3, carry):
                @pl.when(g3 * NBUF < nch)
                def _():
                    descs = []
                    for b in range(NBUF):
                        ch = g3 * NBUF + b
                        descs.append(pltpu.async_copy(
                            h_hbm.at[src_f.at[ch]], bufs[b], sem))
                    for b in range(NBUF):
                        descs[b].wait()
                    for b in range(NBUF):
                        pltpu.sync_copy(
                            bufs[b], acc.at[dst2d.at[g3 * NBUF + b]],
                            add=True)
                return carry

            lax.fori_loop(0, nchcap // NBUF, group, 0)

        plsc.subcore_barrier()
        pltpu.sync_copy(acc.at[pl.ds(s * orows, orows)],
                        out_hbm.at[c, pl.ds(s * orows, orows)])

    return agg


def _mlp_body(scale_ref, h_ref, agg_ref, w1_ref, b1_ref, g_ref, be_ref, w2_ref,
              b2_ref, out_ref):
    n = h_ref.shape[0]
    aggcat = jnp.concatenate([agg_ref[0], agg_ref[1]], axis=0)
    z = h_ref[:] * scale_ref[0, 0] + aggcat[:n]
    z1 = jnp.dot(z, w1_ref[:], preferred_element_type=jnp.float32) + b1_ref[:]
    z1 = jnp.maximum(z1, 0.0)
    mu = jnp.mean(z1, axis=0, keepdims=True)
    cen = z1 - mu
    var = jnp.mean(cen * cen, axis=0, keepdims=True)
    z2 = cen * lax.rsqrt(var + 1e-5) * g_ref[:] + be_ref[:]
    z2 = jnp.maximum(z2, 0.0)
    z3 = jnp.dot(z2, w2_ref[:], preferred_element_type=jnp.float32) + b2_ref[:]
    out_ref[:] = jnp.maximum(z3, 0.0)


@functools.cache
def _mlp_call(n, d, h):
    return pl.pallas_call(
        _mlp_body,
        out_shape=jax.ShapeDtypeStruct((n, h), jnp.float32),
        in_specs=[pl.BlockSpec(memory_space=pltpu.SMEM)]
        + [pl.BlockSpec(memory_space=pltpu.VMEM)] * 8,
        out_specs=pl.BlockSpec(memory_space=pltpu.VMEM),
    )


def _final_body(batch_ref, h_ref, w1_ref, b1_ref, w2_ref, b2_ref, out_ref, *,
                g):
    n = h_ref.shape[0]
    gids = lax.broadcasted_iota(jnp.int32, (g, n), 0)
    onehot = (batch_ref[:] == gids).astype(jnp.float32)
    gp = jnp.dot(onehot, h_ref[:], preferred_element_type=jnp.float32)
    g1 = jnp.dot(gp, w1_ref[:], preferred_element_type=jnp.float32) + b1_ref[:]
    g1 = jnp.maximum(g1, 0.0)
    out_ref[:] = (jnp.dot(g1, w2_ref[:], preferred_element_type=jnp.float32)
                  + b2_ref[:])


@functools.cache
def _final_call(n, h, o, g):
    return pl.pallas_call(
        functools.partial(_final_body, g=g),
        out_shape=jax.ShapeDtypeStruct((g, o), jnp.float32),
        in_specs=[pl.BlockSpec(memory_space=pltpu.VMEM)] * 6,
        out_specs=pl.BlockSpec(memory_space=pltpu.VMEM),
    )


def kernel(x, edge_index, batch, eps, W1, b1, gamma, beta, W2, b2, lin1_W,
           lin1_b, lin2_W, lin2_b):
    n, d = x.shape
    e = edge_index.shape[1]
    nlayers, _, hdim = W1.shape
    odim = lin2_W.shape[1]
    g = 64

    nw = NC * NS
    epw = e // nw
    epw_pad = ((epw + 15) // 16) * 16
    half = ((n + 2 * NS * 8 - 1) // (2 * NS * 8)) * (NS * 8)
    src = edge_index[0].reshape(nw, epw)
    dst = edge_index[1].reshape(nw, epw)
    pad = epw_pad - epw
    if pad:
        src = jnp.pad(src, ((0, 0), (0, pad)))
        dst = jnp.pad(dst, ((0, 0), (0, pad)), constant_values=half)
    cap = ((epw_pad + NBUF * CH) + NBUF * CH - 1) // (NBUF * CH) * (NBUF * CH)

    esrc, edst, cnts = _part_call(epw_pad, half, cap)(src, dst)
    zeros = jnp.zeros((half + CH, d), jnp.float32)
    agg_fn = _agg_call(n, d, half, cap)
    mlp_fn = _mlp_call(n, d, hdim)

    h = x
    for i in range(nlayers):
        agg = agg_fn(h, esrc, edst, cnts, zeros)
        scale = (1.0 + eps[i]).reshape(1, 1)
        h = mlp_fn(scale, h, agg, W1[i], b1[i].reshape(1, hdim),
                   gamma[i].reshape(1, hdim), beta[i].reshape(1, hdim), W2[i],
                   b2[i].reshape(1, hdim))

    return _final_call(n, hdim, odim, g)(
        batch.reshape(1, n), h, lin1_W, lin1_b.reshape(1, odim), lin2_W,
        lin2_b.reshape(1, odim))


# speedup vs baseline: 3.7444x; 1.1621x over previous
import functools

import jax
import jax.numpy as jnp
from jax import lax
from jax.experimental import pallas as pl
from jax.experimental.pallas import tpu as pltpu
import jax.experimental.pallas.tpu_sc as plsc

NC = 2
NS = 16
CH = 128
NBUF = 3


@functools.cache
def _part_call(epw_pad, half, cap):
    nw = NC * NS
    nchcap = cap // CH
    assert cap >= epw_pad + NBUF * CH

    mesh = plsc.VectorSubcoreMesh(core_axis_name="c", subcore_axis_name="s")

    @functools.partial(
        pl.kernel,
        out_type=[
            jax.ShapeDtypeStruct((2, nw, nchcap, CH), jnp.int32),
            jax.ShapeDtypeStruct((2, nw, nchcap, CH), jnp.int32),
            jax.ShapeDtypeStruct((2, nw, 16), jnp.int32),
        ],
        mesh=mesh,
        compiler_params=pltpu.CompilerParams(needs_layout_passes=False),
        scratch_types=[
            pltpu.VMEM((epw_pad,), jnp.int32),
            pltpu.VMEM((epw_pad,), jnp.int32),
            pltpu.VMEM((cap,), jnp.int32),
            pltpu.VMEM((cap,), jnp.int32),
            pltpu.VMEM((cap,), jnp.int32),
            pltpu.VMEM((cap,), jnp.int32),
            pltpu.VMEM((16,), jnp.int32),
            pltpu.VMEM((16,), jnp.int32),
        ],
    )
    def part(srcp_hbm, dstp_hbm, esrc_hbm, edst_hbm, cnts_hbm, src_f, dst_f,
             lo_src, lo_dst, hi_src, hi_dst, cnt_lo, cnt_hi):
        c = lax.axis_index("c")
        s = lax.axis_index("s")
        wid = c * NS + s
        pltpu.sync_copy(srcp_hbm.at[wid], src_f)
        pltpu.sync_copy(dstp_hbm.at[wid], dst_f)

        iota16 = lax.broadcasted_iota(jnp.int32, (16,), 0)
        trash = cap - 16 + iota16
        zero16 = jnp.zeros((16,), jnp.int32)
        padrow = jnp.full((16,), half, jnp.int32)

        def prefill(j, carry):
            spread = lax.rem(j * 16 + iota16, 1024)
            lo_src[pl.ds(j * 16, 16)] = spread
            lo_dst[pl.ds(j * 16, 16)] = padrow
            hi_src[pl.ds(j * 16, 16)] = spread
            hi_dst[pl.ds(j * 16, 16)] = padrow
            return carry

        lax.fori_loop(0, cap // 16, prefill, 0)

        def step(i, carry):
            clo, chi = carry
            sv = src_f[pl.ds(i * 16, 16)]
            dv = dst_f[pl.ds(i * 16, 16)]
            mlo = dv < half
            mi = mlo.astype(jnp.int32)
            dloc = jnp.where(mlo, dv, dv - half)
            exc = plsc.cumsum(mi) - mi
            plo = jnp.where(mlo, clo + exc, trash)
            phi = jnp.where(mlo, trash, chi + iota16 - exc)
            plsc.store_scatter(lo_src, [plo], sv)
            plsc.store_scatter(lo_dst, [plo], dloc)
            plsc.store_scatter(hi_src, [phi], sv)
            plsc.store_scatter(hi_dst, [phi], dloc)
            nlo = jnp.sum(mi)
            return clo + nlo, chi + (16 - nlo)

        clo, chi = lax.fori_loop(0, epw_pad // 16, step, (0, 0))


        for r in range(nchcap):
            pltpu.sync_copy(lo_src.at[pl.ds(r * CH, CH)],
                            esrc_hbm.at[0, wid, r])
            pltpu.sync_copy(hi_src.at[pl.ds(r * CH, CH)],
                            esrc_hbm.at[1, wid, r])
            pltpu.sync_copy(lo_dst.at[pl.ds(r * CH, CH)],
                            edst_hbm.at[0, wid, r])
            pltpu.sync_copy(hi_dst.at[pl.ds(r * CH, CH)],
                            edst_hbm.at[1, wid, r])

        ncl = lax.div(clo + (CH - 1), CH)
        ncl = lax.div(ncl + (NBUF - 1), NBUF) * NBUF
        nch = lax.div(chi + (CH - 1), CH)
        nch = lax.div(nch + (NBUF - 1), NBUF) * NBUF
        cnt_lo[...] = jnp.full((16,), ncl, jnp.int32)
        cnt_hi[...] = jnp.full((16,), nch, jnp.int32)
        pltpu.sync_copy(cnt_lo, cnts_hbm.at[0, wid])
        pltpu.sync_copy(cnt_hi, cnts_hbm.at[1, wid])

    return part


@functools.cache
def _agg_call(n, d, half, cap):
    nw = NC * NS
    nchcap = cap // CH
    acc_rows = half + CH
    assert acc_rows % (NS * 8) == 0 and half % (NS * 8) == 0
    zrows = acc_rows // NS
    orows = half // NS

    mesh = plsc.VectorSubcoreMesh(core_axis_name="c", subcore_axis_name="s")

    @functools.partial(
        pl.kernel,
        out_type=jax.ShapeDtypeStruct((2, half, d), jnp.float32),
        mesh=mesh,
        compiler_params=pltpu.CompilerParams(needs_layout_passes=False),
        scratch_types=(
            [pltpu.VMEM((nchcap, CH), jnp.int32),
             pltpu.VMEM((nchcap, CH), jnp.int32),
             pltpu.VMEM((16,), jnp.int32),
             pltpu.VMEM((16,), jnp.int32)]
            + [pltpu.VMEM((CH, d), jnp.float32) for _ in range(NBUF)]
            + [pltpu.VMEM_SHARED((acc_rows, d), jnp.float32),
               pltpu.SemaphoreType.DMA, pltpu.SemaphoreType.DMA]
        ),
    )
    def agg(h_hbm, esrc_hbm, edst_hbm, cnts_hbm, zeros_hbm, out_hbm, src_f,
            dst2d, cnts_v0, cnts_v1, *rest):
        bufs = rest[:NBUF]
        acc = rest[NBUF]
        sem = rest[NBUF + 1]
        sem_s = rest[NBUF + 2]
        c = lax.axis_index("c")
        s = lax.axis_index("s")
        pltpu.sync_copy(zeros_hbm.at[pl.ds(s * zrows, zrows)],
                        acc.at[pl.ds(s * zrows, zrows)])
        pltpu.sync_copy(cnts_hbm.at[c, 2 * s], cnts_v0)
        pltpu.sync_copy(cnts_hbm.at[c, 2 * s + 1], cnts_v1)
        plsc.subcore_barrier()
        nch_w = [jnp.max(cnts_v0[...]), jnp.max(cnts_v1[...])]

        for wsub in range(2):
            w = 2 * s + wsub
            pltpu.sync_copy(esrc_hbm.at[c, w], src_f)
            pltpu.sync_copy(edst_hbm.at[c, w], dst2d)

            nch = nch_w[wsub]

            def group(g3, carry):
                @pl.when(g3 * NBUF < nch)
                def _():
                    descs = []
                    for b in range(NBUF):
                        ch = g3 * NBUF + b
                        descs.append(pltpu.async_copy(
                            h_hbm.at[src_f.at[ch]], bufs[b], sem))
                    sdescs = []
                    for b in range(NBUF):
                        descs[b].wait()
                        sdescs.append(pltpu.async_copy(
                            bufs[b], acc.at[dst2d.at[g3 * NBUF + b]], sem_s,
                            add=True))
                    for b in range(NBUF):
                        sdescs[b].wait()
                return carry

            lax.fori_loop(0, nchcap // NBUF, group, 0)

        plsc.subcore_barrier()
        pltpu.sync_copy(acc.at[pl.ds(s * orows, orows)],
                        out_hbm.at[c, pl.ds(s * orows, orows)])

    return agg


def _mlp_body(scale_ref, h_ref, agg_ref, w1_ref, b1_ref, g_ref, be_ref, w2_ref,
              b2_ref, out_ref):
    n = h_ref.shape[0]
    aggcat = jnp.concatenate([agg_ref[0], agg_ref[1]], axis=0)
    z = h_ref[:] * scale_ref[0, 0] + aggcat[:n]
    z1 = jnp.dot(z, w1_ref[:], preferred_element_type=jnp.float32) + b1_ref[:]
    z1 = jnp.maximum(z1, 0.0)
    mu = jnp.mean(z1, axis=0, keepdims=True)
    cen = z1 - mu
    var = jnp.mean(cen * cen, axis=0, keepdims=True)
    z2 = cen * lax.rsqrt(var + 1e-5) * g_ref[:] + be_ref[:]
    z2 = jnp.maximum(z2, 0.0)
    z3 = jnp.dot(z2, w2_ref[:], preferred_element_type=jnp.float32) + b2_ref[:]
    out_ref[:] = jnp.maximum(z3, 0.0)


@functools.cache
def _mlp_call(n, d, h):
    return pl.pallas_call(
        _mlp_body,
        out_shape=jax.ShapeDtypeStruct((n, h), jnp.float32),
        in_specs=[pl.BlockSpec(memory_space=pltpu.SMEM)]
        + [pl.BlockSpec(memory_space=pltpu.VMEM)] * 8,
        out_specs=pl.BlockSpec(memory_space=pltpu.VMEM),
    )


def _final_body(batch_ref, h_ref, w1_ref, b1_ref, w2_ref, b2_ref, out_ref, *,
                g):
    n = h_ref.shape[0]
    gids = lax.broadcasted_iota(jnp.int32, (g, n), 0)
    onehot = (batch_ref[:] == gids).astype(jnp.float32)
    gp = jnp.dot(onehot, h_ref[:], preferred_element_type=jnp.float32)
    g1 = jnp.dot(gp, w1_ref[:], preferred_element_type=jnp.float32) + b1_ref[:]
    g1 = jnp.maximum(g1, 0.0)
    out_ref[:] = (jnp.dot(g1, w2_ref[:], preferred_element_type=jnp.float32)
                  + b2_ref[:])


@functools.cache
def _final_call(n, h, o, g):
    return pl.pallas_call(
        functools.partial(_final_body, g=g),
        out_shape=jax.ShapeDtypeStruct((g, o), jnp.float32),
        in_specs=[pl.BlockSpec(memory_space=pltpu.VMEM)] * 6,
        out_specs=pl.BlockSpec(memory_space=pltpu.VMEM),
    )


def kernel(x, edge_index, batch, eps, W1, b1, gamma, beta, W2, b2, lin1_W,
           lin1_b, lin2_W, lin2_b):
    n, d = x.shape
    e = edge_index.shape[1]
    nlayers, _, hdim = W1.shape
    odim = lin2_W.shape[1]
    g = 64

    nw = NC * NS
    epw = e // nw
    epw_pad = ((epw + 15) // 16) * 16
    half = ((n + 2 * NS * 8 - 1) // (2 * NS * 8)) * (NS * 8)
    src = edge_index[0].reshape(nw, epw)
    dst = edge_index[1].reshape(nw, epw)
    pad = epw_pad - epw
    if pad:
        src = jnp.pad(src, ((0, 0), (0, pad)))
        dst = jnp.pad(dst, ((0, 0), (0, pad)), constant_values=half)
    cap = ((epw_pad + NBUF * CH) + NBUF * CH - 1) // (NBUF * CH) * (NBUF * CH)

    esrc, edst, cnts = _part_call(epw_pad, half, cap)(src, dst)
    zeros = jnp.zeros((half + CH, d), jnp.float32)
    agg_fn = _agg_call(n, d, half, cap)
    mlp_fn = _mlp_call(n, d, hdim)

    h = x
    for i in range(nlayers):
        agg = agg_fn(h, esrc, edst, cnts, zeros)
        scale = (1.0 + eps[i]).reshape(1, 1)
        h = mlp_fn(scale, h, agg, W1[i], b1[i].reshape(1, hdim),
                   gamma[i].reshape(1, hdim), beta[i].reshape(1, hdim), W2[i],
                   b2[i].reshape(1, hdim))

    return _final_call(n, hdim, odim, g)(
        batch.reshape(1, n), h, lin1_W, lin1_b.reshape(1, odim), lin2_W,
        lin2_b.reshape(1, odim))
